# baseline (speedup 1.0000x reference)
import functools

import jax
import jax.numpy as jnp
from jax.experimental import pallas as pl
from jax.experimental.pallas import tpu as pltpu

_NBUF = 8
_CHUNK = 512


def _route_rows(xb, w, b, ec):
    proj = jax.lax.dot_general(
        xb, w, (((1,), (1,)), ((), ())),
        precision=jax.lax.Precision.DEFAULT,
        preferred_element_type=jnp.float32,
    )
    proj = proj + b
    n = jnp.sqrt(jnp.sum(proj * proj, axis=1, keepdims=True))
    projn = proj / jnp.maximum(n, 1e-12)

    cos = jax.lax.dot_general(
        projn, ec, (((1,), (1,)), ((), ())),
        precision=jax.lax.Precision.DEFAULT,
        preferred_element_type=jnp.float32,
    )

    C = cos.shape[1]
    iota = jax.lax.broadcasted_iota(jnp.int32, cos.shape, 1).astype(jnp.float32)
    m1 = jnp.max(cos, axis=1, keepdims=True)
    i1 = jnp.min(jnp.where(cos == m1, iota, float(C)), axis=1, keepdims=True)
    cos2 = jnp.where(iota == i1, -jnp.inf, cos)
    m2 = jnp.max(cos2, axis=1, keepdims=True)
    i2 = jnp.min(jnp.where(cos2 == m2, iota, float(C)), axis=1, keepdims=True)

    e = jnp.exp(m2 - m1)
    denom = 1.0 + e
    p1 = 1.0 / denom
    p2 = e / denom
    probs = jnp.concatenate([p1, p2], axis=1).T
    idx = jnp.concatenate([i1, i2], axis=1).T.astype(jnp.int32)
    return probs, idx


def _router_kernel(x_hbm, ec_ref, w_ref, b_ref, probs_ref, idx_ref,
                   xbuf, sems):
    M = x_hbm.shape[0]
    nchunk = M // _CHUNK

    w = w_ref[...]
    b = b_ref[...]
    ec = ec_ref[...]
    ecn = jnp.sqrt(jnp.sum(ec * ec, axis=1, keepdims=True))
    ecn = ec / jnp.maximum(ecn, 1e-12)

    def copy_in(chunk, slot):
        pltpu.make_async_copy(
            x_hbm.at[pl.ds(chunk * _CHUNK, _CHUNK), :],
            xbuf.at[slot],
            sems.at[slot],
        ).start()

    for j in range(min(_NBUF, nchunk)):
        copy_in(j, j)

    def body(i, carry):
        slot = jax.lax.rem(i, _NBUF)
        pltpu.make_async_copy(
            x_hbm.at[pl.ds(i * _CHUNK, _CHUNK), :],
            xbuf.at[slot],
            sems.at[slot],
        ).wait()
        p, ix = _route_rows(xbuf[slot], w, b, ecn)
        probs_ref[:, pl.ds(i * _CHUNK, _CHUNK)] = p
        idx_ref[:, pl.ds(i * _CHUNK, _CHUNK)] = ix

        @pl.when(i + _NBUF < nchunk)
        def _():
            copy_in(i + _NBUF, slot)

        return carry

    jax.lax.fori_loop(0, nchunk, body, 0)


@functools.partial(jax.jit, static_argnames=())
def kernel(x, expert_centers, W, b):
    bs, C, T2 = x.shape
    E = W.shape[0]
    M = bs * C
    x2 = x.reshape(M, T2)
    b2 = b.reshape(1, E)
    probs2, idx2 = pl.pallas_call(
        _router_kernel,
        in_specs=[
            pl.BlockSpec(memory_space=pl.ANY),
            pl.BlockSpec(memory_space=pltpu.VMEM),
            pl.BlockSpec(memory_space=pltpu.VMEM),
            pl.BlockSpec(memory_space=pltpu.VMEM),
        ],
        out_specs=[
            pl.BlockSpec(memory_space=pltpu.VMEM),
            pl.BlockSpec(memory_space=pltpu.VMEM),
        ],
        out_shape=[
            jax.ShapeDtypeStruct((2, M), jnp.float32),
            jax.ShapeDtypeStruct((2, M), jnp.int32),
        ],
        scratch_shapes=[
            pltpu.VMEM((_NBUF, _CHUNK, T2), jnp.float32),
            pltpu.SemaphoreType.DMA((_NBUF,)),
        ],
    )(x2, expert_centers, W, b2)
    return (probs2.T.reshape(bs, C, 2), idx2.T.reshape(bs, C, 2))


# speedup vs baseline: 1.1983x; 1.1983x over previous
import functools

import jax
import jax.numpy as jnp
from jax.experimental import pallas as pl
from jax.experimental.pallas import tpu as pltpu

_NBUF = 3
_CHUNK = 2048


def _route_rows(xb, w, b, ec):
    proj = jax.lax.dot_general(
        xb, w, (((1,), (1,)), ((), ())),
        precision=jax.lax.Precision.DEFAULT,
        preferred_element_type=jnp.float32,
    )
    proj = proj + b
    n = jnp.sqrt(jnp.sum(proj * proj, axis=1, keepdims=True))
    projn = proj / jnp.maximum(n, 1e-12)

    cos = jax.lax.dot_general(
        projn, ec, (((1,), (1,)), ((), ())),
        precision=jax.lax.Precision.DEFAULT,
        preferred_element_type=jnp.float32,
    )

    C = cos.shape[1]
    iota = jax.lax.broadcasted_iota(jnp.int32, cos.shape, 1).astype(jnp.float32)
    m1 = jnp.max(cos, axis=1, keepdims=True)
    i1 = jnp.min(jnp.where(cos == m1, iota, float(C)), axis=1, keepdims=True)
    cos2 = jnp.where(iota == i1, -jnp.inf, cos)
    m2 = jnp.max(cos2, axis=1, keepdims=True)
    i2 = jnp.min(jnp.where(cos2 == m2, iota, float(C)), axis=1, keepdims=True)

    e = jnp.exp(m2 - m1)
    denom = 1.0 + e
    p1 = 1.0 / denom
    p2 = e / denom
    probs = jnp.concatenate([p1, p2], axis=1).T
    idx = jnp.concatenate([i1, i2], axis=1).T.astype(jnp.int32)
    return probs, idx


def _router_kernel(x_hbm, ec_ref, w_ref, b_ref, probs_ref, idx_ref,
                   xbuf, sems):
    M = x_hbm.shape[0]
    nchunk = M // _CHUNK

    w = w_ref[...]
    b = b_ref[...]
    ec = ec_ref[...]
    ecn = jnp.sqrt(jnp.sum(ec * ec, axis=1, keepdims=True))
    ecn = ec / jnp.maximum(ecn, 1e-12)

    def copy_in(chunk, slot):
        pltpu.make_async_copy(
            x_hbm.at[pl.ds(chunk * _CHUNK, _CHUNK), :],
            xbuf.at[slot],
            sems.at[slot],
        ).start()

    for j in range(min(_NBUF, nchunk)):
        copy_in(j, j)

    def body(i, carry):
        slot = jax.lax.rem(i, _NBUF)
        pltpu.make_async_copy(
            x_hbm.at[pl.ds(i * _CHUNK, _CHUNK), :],
            xbuf.at[slot],
            sems.at[slot],
        ).wait()
        p, ix = _route_rows(xbuf[slot], w, b, ecn)
        probs_ref[:, pl.ds(i * _CHUNK, _CHUNK)] = p
        idx_ref[:, pl.ds(i * _CHUNK, _CHUNK)] = ix

        @pl.when(i + _NBUF < nchunk)
        def _():
            copy_in(i + _NBUF, slot)

        return carry

    jax.lax.fori_loop(0, nchunk, body, 0)


@functools.partial(jax.jit, static_argnames=())
def kernel(x, expert_centers, W, b):
    bs, C, T2 = x.shape
    E = W.shape[0]
    M = bs * C
    x2 = x.reshape(M, T2)
    b2 = b.reshape(1, E)
    probs2, idx2 = pl.pallas_call(
        _router_kernel,
        in_specs=[
            pl.BlockSpec(memory_space=pl.ANY),
            pl.BlockSpec(memory_space=pltpu.VMEM),
            pl.BlockSpec(memory_space=pltpu.VMEM),
            pl.BlockSpec(memory_space=pltpu.VMEM),
        ],
        out_specs=[
            pl.BlockSpec(memory_space=pltpu.VMEM),
            pl.BlockSpec(memory_space=pltpu.VMEM),
        ],
        out_shape=[
            jax.ShapeDtypeStruct((2, M), jnp.float32),
            jax.ShapeDtypeStruct((2, M), jnp.int32),
        ],
        scratch_shapes=[
            pltpu.VMEM((_NBUF, _CHUNK, T2), jnp.float32),
            pltpu.SemaphoreType.DMA((_NBUF,)),
        ],
    )(x2, expert_centers, W, b2)
    return (probs2.T.reshape(bs, C, 2), idx2.T.reshape(bs, C, 2))
